# baseline (speedup 1.0000x reference)
import functools

import jax
import jax.numpy as jnp
from jax import lax
from jax.experimental import pallas as pl
from jax.experimental.pallas import tpu as pltpu
from jax.experimental.pallas import tpu_sc as plsc


def _tc_add_kernel(x_ref, pe_ref, o_ref):
    o_ref[...] = x_ref[...] + pe_ref[...]


@jax.jit
def _kernel_tc(x, pe_table):
    B, S, D = x.shape
    R = 2048

    grid = (S // R, B)

    return pl.pallas_call(
        _tc_add_kernel,
        grid=grid,
        in_specs=[
            pl.BlockSpec((1, R, D), lambda i, j: (j, i, 0)),
            pl.BlockSpec((R, D), lambda i, j: (i, 0)),
        ],
        out_specs=pl.BlockSpec((1, R, D), lambda i, j: (j, i, 0)),
        out_shape=jax.ShapeDtypeStruct((B, S, D), x.dtype),
        compiler_params=pltpu.CompilerParams(
            dimension_semantics=("arbitrary", "arbitrary"),
        ),
    )(x, pe_table)


_NW = 32
_ROWS_PER_W = 64
_XB = 16


def _make_sc_add(B, S, D):
    n_pe_chunks = _ROWS_PER_W // _XB
    n_chunks = n_pe_chunks * B

    mesh = plsc.VectorSubcoreMesh(core_axis_name="c", subcore_axis_name="s")

    NB = 3

    @functools.partial(
        pl.kernel,
        mesh=mesh,
        out_type=jax.ShapeDtypeStruct((B * S, D), jnp.float32),
        scratch_types=[
            pltpu.VMEM((NB, _XB, D), jnp.float32),
            pltpu.VMEM((_ROWS_PER_W, D), jnp.float32),
            pltpu.SemaphoreType.DMA((NB,)),
            pltpu.SemaphoreType.DMA((NB,)),
            pltpu.SemaphoreType.DMA,
        ],
    )
    def sc_add(x_hbm, pe_hbm, out_hbm, x_v, pe_v, in_sem, out_sem, pe_sem):
        c = lax.axis_index("c")
        s = lax.axis_index("s")
        wid = s * 2 + c
        pe_row0 = wid * _ROWS_PER_W

        def row0(k):
            p, b = divmod(k, B)
            return b * S + pe_row0 + p * _XB

        def start_in(k):
            return pltpu.async_copy(
                x_hbm.at[pl.ds(row0(k), _XB)], x_v.at[k % NB],
                in_sem.at[k % NB])

        pe_cp = pltpu.async_copy(
            pe_hbm.at[pl.ds(pe_row0, _ROWS_PER_W)], pe_v, pe_sem)
        in_cp = {k: start_in(k) for k in range(NB)}
        out_cp = {}
        out_waited = set()

        pe_cp.wait()
        for k in range(n_chunks):
            p = k // B
            in_cp[k].wait()

            xb = x_v.at[k % NB]
            prow = p * _XB

            @plsc.parallel_loop(0, _XB * D, 16, unroll=8)
            def _(i):
                r = i // D
                col = i - r * D
                sl = pl.ds(col, 16)
                xb[r, sl] = xb[r, sl] + pe_v[prow + r, sl]

            out_cp[k] = pltpu.async_copy(
                xb, out_hbm.at[pl.ds(row0(k), _XB)], out_sem.at[k % NB])
            if k >= 1 and k + NB - 1 < n_chunks:
                out_cp[k - 1].wait()
                out_waited.add(k - 1)
                in_cp[k + NB - 1] = start_in(k + NB - 1)

        for k in range(n_chunks):
            if k not in out_waited:
                out_cp[k].wait()

    return sc_add


@jax.jit
def _kernel_sc(x, pe_table):
    B, S, D = x.shape
    out = _make_sc_add(B, S, D)(x.reshape(B * S, D), pe_table)
    return out.reshape(B, S, D)


@jax.jit
def _kernel_hybrid(x, pe_table):
    B, S, D = x.shape
    sc = _make_sc_add(1, S, D)(x[B - 1], pe_table).reshape(1, S, D)
    tc = _kernel_tc(x[: B - 1], pe_table)
    return jnp.concatenate([tc, sc], axis=0)


kernel = _kernel_sc


# speedup vs baseline: 1.0407x; 1.0407x over previous
import functools

import jax
import jax.numpy as jnp
from jax import lax
from jax.experimental import pallas as pl
from jax.experimental.pallas import tpu as pltpu
from jax.experimental.pallas import tpu_sc as plsc


def _tc_add_kernel(x_ref, pe_ref, o_ref):
    o_ref[...] = x_ref[...] + pe_ref[...]


@jax.jit
def _kernel_tc(x, pe_table):
    B, S, D = x.shape
    R = 2048

    grid = (S // R, B)

    return pl.pallas_call(
        _tc_add_kernel,
        grid=grid,
        in_specs=[
            pl.BlockSpec((1, R, D), lambda i, j: (j, i, 0)),
            pl.BlockSpec((R, D), lambda i, j: (i, 0)),
        ],
        out_specs=pl.BlockSpec((1, R, D), lambda i, j: (j, i, 0)),
        out_shape=jax.ShapeDtypeStruct((B, S, D), x.dtype),
        compiler_params=pltpu.CompilerParams(
            dimension_semantics=("arbitrary", "arbitrary"),
        ),
    )(x, pe_table)


_NW = 32
_ROWS_PER_W = 64
_XB = 16


def _make_sc_add(B, S, D):
    n_pe_chunks = _ROWS_PER_W // _XB
    n_chunks = n_pe_chunks * B

    mesh = plsc.VectorSubcoreMesh(core_axis_name="c", subcore_axis_name="s")

    NB = 4

    @functools.partial(
        pl.kernel,
        mesh=mesh,
        out_type=jax.ShapeDtypeStruct((B * S, D), jnp.float32),
        scratch_types=[
            pltpu.VMEM((NB, _XB, D), jnp.float32),
            pltpu.VMEM((2, _XB, D), jnp.float32),
            pltpu.SemaphoreType.DMA((NB,)),
            pltpu.SemaphoreType.DMA((NB,)),
            pltpu.SemaphoreType.DMA((2,)),
        ],
    )
    def sc_add(x_hbm, pe_hbm, out_hbm, x_v, pe_v, in_sem, out_sem, pe_sem):
        c = lax.axis_index("c")
        s = lax.axis_index("s")
        wid = s * 2 + c
        pe_row0 = wid * _ROWS_PER_W

        def row0(k):
            p, b = divmod(k, B)
            return b * S + pe_row0 + p * _XB

        def start_in(k):
            return pltpu.async_copy(
                x_hbm.at[pl.ds(row0(k), _XB)], x_v.at[k % NB],
                in_sem.at[k % NB])

        def start_pe(p):
            return pltpu.async_copy(
                pe_hbm.at[pl.ds(pe_row0 + p * _XB, _XB)], pe_v.at[p % 2],
                pe_sem.at[p % 2])

        in_cp = {k: start_in(k) for k in range(NB)}
        pe_cp = {p: start_pe(p) for p in range(2)}
        out_cp = {}
        out_waited = set()

        for k in range(n_chunks):
            p = k // B
            if k % B == 0:
                pe_cp[p].wait()
            in_cp[k].wait()

            xb, pb = x_v.at[k % NB], pe_v.at[p % 2]

            @plsc.parallel_loop(0, _XB * D, 16, unroll=8)
            def _(i):
                r = i // D
                col = i - r * D
                sl = pl.ds(col, 16)
                xb[r, sl] = xb[r, sl] + pb[r, sl]

            out_cp[k] = pltpu.async_copy(
                xb, out_hbm.at[pl.ds(row0(k), _XB)], out_sem.at[k % NB])
            if k >= 1 and k + NB - 1 < n_chunks:
                out_cp[k - 1].wait()
                out_waited.add(k - 1)
                in_cp[k + NB - 1] = start_in(k + NB - 1)
            if k % B == B - 1 and p + 2 < n_pe_chunks:
                pe_cp[p + 2] = start_pe(p + 2)

        for k in range(n_chunks):
            if k not in out_waited:
                out_cp[k].wait()

    return sc_add


@jax.jit
def _kernel_sc(x, pe_table):
    B, S, D = x.shape
    out = _make_sc_add(B, S, D)(x.reshape(B * S, D), pe_table)
    return out.reshape(B, S, D)


@jax.jit
def _kernel_hybrid(x, pe_table):
    B, S, D = x.shape
    sc = _make_sc_add(1, S, D)(x[B - 1], pe_table).reshape(1, S, D)
    tc = _kernel_tc(x[: B - 1], pe_table)
    return jnp.concatenate([tc, sc], axis=0)


kernel = _kernel_sc


# speedup vs baseline: 1.0415x; 1.0008x over previous
import functools

import jax
import jax.numpy as jnp
from jax import lax
from jax.experimental import pallas as pl
from jax.experimental.pallas import tpu as pltpu
from jax.experimental.pallas import tpu_sc as plsc


_NW = 32
_ROWS_PER_W = 64
_XB = 16


def _make_sc_add(B, S, D):
    n_pe_chunks = _ROWS_PER_W // _XB
    n_chunks = n_pe_chunks * B

    mesh = plsc.VectorSubcoreMesh(core_axis_name="c", subcore_axis_name="s")

    NB = 4

    @functools.partial(
        pl.kernel,
        mesh=mesh,
        out_type=jax.ShapeDtypeStruct((B * S, D), jnp.float32),
        scratch_types=[
            pltpu.VMEM((NB, _XB, D), jnp.float32),
            pltpu.VMEM((2, _XB, D), jnp.float32),
            pltpu.SemaphoreType.DMA((NB,)),
            pltpu.SemaphoreType.DMA((NB,)),
            pltpu.SemaphoreType.DMA((2,)),
        ],
    )
    def sc_add(x_hbm, pe_hbm, out_hbm, x_v, pe_v, in_sem, out_sem, pe_sem):
        c = lax.axis_index("c")
        s = lax.axis_index("s")
        wid = s * 2 + c
        pe_row0 = wid * _ROWS_PER_W

        def row0(k):
            p, b = divmod(k, B)
            return b * S + pe_row0 + p * _XB

        def start_in(k):
            return pltpu.async_copy(
                x_hbm.at[pl.ds(row0(k), _XB)], x_v.at[k % NB],
                in_sem.at[k % NB])

        def start_pe(p):
            return pltpu.async_copy(
                pe_hbm.at[pl.ds(pe_row0 + p * _XB, _XB)], pe_v.at[p % 2],
                pe_sem.at[p % 2])

        in_cp = {k: start_in(k) for k in range(NB)}
        pe_cp = {p: start_pe(p) for p in range(2)}
        out_cp = {}
        out_waited = set()

        for k in range(n_chunks):
            p = k // B
            if k % B == 0:
                pe_cp[p].wait()
            in_cp[k].wait()

            xb, pb = x_v.at[k % NB], pe_v.at[p % 2]

            @plsc.parallel_loop(0, _XB * D, 16, unroll=8)
            def _(i):
                r = i // D
                col = i - r * D
                sl = pl.ds(col, 16)
                xb[r, sl] = xb[r, sl] + pb[r, sl]

            out_cp[k] = pltpu.async_copy(
                xb, out_hbm.at[pl.ds(row0(k), _XB)], out_sem.at[k % NB])
            if k >= 1 and k + NB - 1 < n_chunks:
                out_cp[k - 1].wait()
                out_waited.add(k - 1)
                in_cp[k + NB - 1] = start_in(k + NB - 1)
            if k % B == B - 1 and p + 2 < n_pe_chunks:
                pe_cp[p + 2] = start_pe(p + 2)

        for k in range(n_chunks):
            if k not in out_waited:
                out_cp[k].wait()

    return sc_add


@jax.jit
def kernel(x, pe_table):
    B, S, D = x.shape
    out = _make_sc_add(B, S, D)(x.reshape(B * S, D), pe_table)
    return out.reshape(B, S, D)
